# baseline (speedup 1.0000x reference)
import functools

import jax
import jax.numpy as jnp
from jax import lax
from jax.experimental import pallas as pl
from jax.experimental.pallas import tpu as pltpu
from jax.experimental.pallas import tpu_sc as plsc

_B, _F, _D, _V = 4096, 26, 64, 100000


def _sc_gather_t(idx, table_t):
    info = plsc.get_sparse_core_info()
    nc, ns = info.num_cores, info.num_subcores

    mesh = plsc.VectorSubcoreMesh(core_axis_name="c", subcore_axis_name="s")

    @functools.partial(
        pl.kernel,
        out_type=jax.ShapeDtypeStruct((_F, _B), jnp.float32),
        mesh=mesh,
        compiler_params=pltpu.CompilerParams(
            needs_layout_passes=False,
            skip_device_barrier=True,
            disable_bounds_checks=True,
            disable_semaphore_checks=True,
        ),
        scratch_types=[
            pltpu.VMEM((_V,), jnp.float32),
            pltpu.VMEM((_B,), jnp.int32),
            pltpu.VMEM((_B,), jnp.float32),
            pltpu.SemaphoreType.DMA,
        ],
    )
    def gather_kernel(idx_hbm, table_hbm, out_hbm, row_v, idx_v, out_v, sem):
        wid = lax.axis_index("s") * nc + lax.axis_index("c")

        @pl.when(wid < _F)
        def _():
            pltpu.async_copy(table_hbm.at[wid], row_v, sem)
            pltpu.sync_copy(idx_hbm, idx_v)
            pltpu.make_async_copy(table_hbm.at[0], row_v, sem).wait()

            def gather_group(g, carry):
                for j in range(8):
                    o = g * 128 + j * 16
                    out_v[pl.ds(o, 16)] = plsc.load_gather(
                        row_v, [idx_v[pl.ds(o, 16)]]
                    )
                return carry

            lax.fori_loop(0, _B // 128, gather_group, 0)
            pltpu.sync_copy(out_v, out_hbm.at[wid])

    return gather_kernel(idx, table_t)


def _tc_pool(alpha_t, stack_t):
    bn = 1024
    grid = (_B // bn,)

    def body(a_ref, x_ref, o_ref):
        a = a_ref[...]
        m = jnp.max(a, axis=0, keepdims=True)
        e = jnp.exp(a - m)
        s = jnp.sum(e, axis=0, keepdims=True)
        w = e / s
        acc = jnp.zeros((_D, bn), jnp.float32)
        for f in range(_F):
            acc = acc + w[f : f + 1, :] * x_ref[f]
        o_ref[...] = acc

    return pl.pallas_call(
        body,
        grid=grid,
        in_specs=[
            pl.BlockSpec((_F, bn), lambda i: (0, i)),
            pl.BlockSpec((_F, _D, bn), lambda i: (0, 0, i)),
        ],
        out_specs=pl.BlockSpec((_D, bn), lambda i: (0, i)),
        out_shape=jax.ShapeDtypeStruct((_D, _B), jnp.float32),
    )(alpha_t, stack_t)


def kernel(stack_embedding, item_input, alpha_embeddings):
    idx = item_input.reshape(-1).astype(jnp.int32)
    table_t = alpha_embeddings.T
    stack_t = jnp.transpose(stack_embedding, (1, 2, 0))
    alpha_t = _sc_gather_t(idx, table_t)
    out_t = _tc_pool(alpha_t, stack_t)
    return out_t.T


# speedup vs baseline: 1.0012x; 1.0012x over previous
import functools

import jax
import jax.numpy as jnp
from jax import lax
from jax.experimental import pallas as pl
from jax.experimental.pallas import tpu as pltpu
from jax.experimental.pallas import tpu_sc as plsc

_B, _F, _D, _V = 4096, 26, 64, 100000


def _sc_gather_t(idx, table_t):
    info = plsc.get_sparse_core_info()
    nc, ns = info.num_cores, info.num_subcores

    mesh = plsc.VectorSubcoreMesh(core_axis_name="c", subcore_axis_name="s")

    @functools.partial(
        pl.kernel,
        out_type=jax.ShapeDtypeStruct((_F, _B), jnp.float32),
        mesh=mesh,
        compiler_params=pltpu.CompilerParams(
            needs_layout_passes=False, skip_device_barrier=True
        ),
        scratch_types=[
            pltpu.VMEM((_V,), jnp.float32),
            pltpu.VMEM((_B,), jnp.int32),
            pltpu.VMEM((_B,), jnp.float32),
            pltpu.SemaphoreType.DMA,
        ],
    )
    def gather_kernel(idx_hbm, table_hbm, out_hbm, row_v, idx_v, out_v, sem):
        wid = lax.axis_index("s") * nc + lax.axis_index("c")

        @pl.when(wid < _F)
        def _():
            pltpu.async_copy(table_hbm.at[wid], row_v, sem)
            pltpu.sync_copy(idx_hbm, idx_v)
            pltpu.make_async_copy(table_hbm.at[0], row_v, sem).wait()

            def gather_group(g, carry):
                for j in range(8):
                    o = g * 128 + j * 16
                    out_v[pl.ds(o, 16)] = plsc.load_gather(
                        row_v, [idx_v[pl.ds(o, 16)]]
                    )
                return carry

            lax.fori_loop(0, _B // 128, gather_group, 0)
            pltpu.sync_copy(out_v, out_hbm.at[wid])

    return gather_kernel(idx, table_t)


def _tc_pool(alpha_t, stack_t):
    bn = 1024
    grid = (_B // bn,)

    def body(a_ref, x_ref, o_ref):
        a = a_ref[...]
        m = jnp.max(a, axis=0, keepdims=True)
        e = jnp.exp(a - m)
        s = jnp.sum(e, axis=0, keepdims=True)
        w = e / s
        acc = jnp.zeros((_D, bn), jnp.float32)
        for f in range(_F):
            acc = acc + w[f : f + 1, :] * x_ref[f]
        o_ref[...] = acc

    return pl.pallas_call(
        body,
        grid=grid,
        in_specs=[
            pl.BlockSpec((_F, bn), lambda i: (0, i)),
            pl.BlockSpec((_F, _D, bn), lambda i: (0, 0, i)),
        ],
        out_specs=pl.BlockSpec((_D, bn), lambda i: (0, i)),
        out_shape=jax.ShapeDtypeStruct((_D, _B), jnp.float32),
    )(alpha_t, stack_t)


def kernel(stack_embedding, item_input, alpha_embeddings):
    idx = item_input.reshape(-1).astype(jnp.int32)
    table_t = alpha_embeddings.T
    stack_t = jnp.transpose(stack_embedding, (1, 2, 0))
    alpha_t = _sc_gather_t(idx, table_t)
    out_t = _tc_pool(alpha_t, stack_t)
    return out_t.T
